# baseline (speedup 1.0000x reference)
import jax
import jax.numpy as jnp
from jax import lax
from jax.experimental import pallas as pl
from jax.experimental.pallas import tpu as pltpu
from jax.experimental.pallas import tpu_sc as plsc

BATCH = 4
SEQ_LEN = 8192
D_MODEL = 1024
FLAT = BATCH * SEQ_LEN

NUM_CORES = 2
NUM_SUBCORES = 16
NW = NUM_CORES * NUM_SUBCORES
S_PER_W = SEQ_LEN // NW
CHUNK = 16
NCHUNK_S = S_PER_W // CHUNK
UNITS = NCHUNK_S * BATCH
GROUPS = CHUNK * D_MODEL // 16
G_PER_ROW = D_MODEL // 16
NBUF = 5


def _body(ids_hbm, tok_hbm, pos_hbm, out_hbm, idx_v, *scratch):
    bufs = scratch[:NBUF]
    pbufs = scratch[NBUF:NBUF + 2]
    gsems = scratch[NBUF + 2:2 * NBUF + 2]
    wsems = scratch[2 * NBUF + 2:3 * NBUF + 2]
    psems = scratch[3 * NBUF + 2:]

    wid = lax.axis_index("s") * NUM_CORES + lax.axis_index("c")
    s_base = wid * S_PER_W

    def start_gather(u):
        c, b = divmod(u, BATCH)
        return pltpu.async_copy(
            tok_hbm.at[idx_v.at[b, pl.ds(c * CHUNK, CHUNK)]],
            bufs[u % NBUF], gsems[u % NBUF])

    def start_pos(c):
        return pltpu.async_copy(pos_hbm.at[pl.ds(s_base + c * CHUNK, CHUNK)],
                                pbufs[c % 2], psems[c % 2])

    pdesc = [start_pos(0), start_pos(1)]

    pltpu.sync_copy(ids_hbm.at[:, pl.ds(s_base, S_PER_W)], idx_v)

    gdesc = [None] * NBUF
    wdesc = [None] * NBUF
    for u in range(NBUF - 2):
        gdesc[u] = start_gather(u)

    for u in range(UNITS):
        slot = u % NBUF
        nu = u + NBUF - 2
        if nu < UNITS:
            ns = nu % NBUF
            if wdesc[ns] is not None:
                wdesc[ns].wait()
                wdesc[ns] = None
            gdesc[ns] = start_gather(nu)

        c, b = divmod(u, BATCH)
        if b == 0:
            pdesc[c % 2].wait()
            pdesc[c % 2] = None
        gdesc[slot].wait()

        cur = bufs[slot]
        pb = pbufs[c % 2]

        @plsc.parallel_loop(0, GROUPS, unroll=16)
        def add_group(i):
            r = lax.div(i, G_PER_ROW)
            off = lax.mul(lax.rem(i, G_PER_ROW), 16)
            plsc.addupdate(cur.at[r, pl.ds(off, 16)], pb[r, pl.ds(off, 16)])

        if b == BATCH - 1 and c + 2 < NCHUNK_S:
            pdesc[c % 2] = start_pos(c + 2)

        row0 = b * SEQ_LEN + s_base + c * CHUNK
        wdesc[slot] = pltpu.async_copy(cur, out_hbm.at[pl.ds(row0, CHUNK)],
                                       wsems[slot])

    for d in wdesc:
        if d is not None:
            d.wait()


@jax.jit
def _embed(ids_r, token_table, pos_table):
    mesh = plsc.VectorSubcoreMesh(core_axis_name="c", subcore_axis_name="s")
    k = pl.kernel(
        _body,
        out_type=jax.ShapeDtypeStruct((FLAT, D_MODEL), jnp.float32),
        mesh=mesh,
        scratch_types=(
            [pltpu.VMEM((BATCH, S_PER_W), jnp.int32)]
            + [pltpu.VMEM((CHUNK, D_MODEL), jnp.float32)] * NBUF
            + [pltpu.VMEM((CHUNK, D_MODEL), jnp.float32)] * 2
            + [pltpu.SemaphoreType.DMA] * (2 * NBUF + 2)
        ),
    )
    return k(ids_r, token_table, pos_table)


def kernel(input_ids, token_table, pos_table):
    out = _embed(input_ids.astype(jnp.int32), token_table, pos_table)
    return out.reshape(BATCH, SEQ_LEN, D_MODEL)


# speedup vs baseline: 1.0157x; 1.0157x over previous
import jax
import jax.numpy as jnp
from jax import lax
from jax.experimental import pallas as pl
from jax.experimental.pallas import tpu as pltpu
from jax.experimental.pallas import tpu_sc as plsc

BATCH = 4
SEQ_LEN = 8192
D_MODEL = 1024
FLAT = BATCH * SEQ_LEN

NUM_CORES = 2
NUM_SUBCORES = 16
NW = NUM_CORES * NUM_SUBCORES
S_PER_W = SEQ_LEN // NW
CHUNK = 16
NCHUNK_S = S_PER_W // CHUNK
UNITS = NCHUNK_S * BATCH
GROUPS = CHUNK * D_MODEL // 16
G_PER_ROW = D_MODEL // 16
NBUF = 5


def _body(ids_hbm, tok_hbm, pos_hbm, out_hbm, idx_v, *scratch):
    bufs = scratch[:NBUF]
    pbufs = scratch[NBUF:NBUF + 2]
    gsems = scratch[NBUF + 2:2 * NBUF + 2]
    wsems = scratch[2 * NBUF + 2:3 * NBUF + 2]
    psems = scratch[3 * NBUF + 2:]

    wid = lax.axis_index("s") * NUM_CORES + lax.axis_index("c")
    s_base = wid * S_PER_W

    def start_gather(u):
        c, b = divmod(u, BATCH)
        return pltpu.async_copy(
            tok_hbm.at[idx_v.at[b, pl.ds(c * CHUNK, CHUNK)]],
            bufs[u % NBUF], gsems[u % NBUF])

    def start_pos(c):
        return pltpu.async_copy(pos_hbm.at[pl.ds(s_base + c * CHUNK, CHUNK)],
                                pbufs[c % 2], psems[c % 2])

    pdesc = [start_pos(0), start_pos(1)]

    pltpu.sync_copy(ids_hbm.at[:, pl.ds(s_base, S_PER_W)], idx_v)

    gdesc = [None] * NBUF
    wdesc = [None] * NBUF
    for u in range(NBUF - 2):
        gdesc[u] = start_gather(u)

    for u in range(UNITS):
        slot = u % NBUF
        nu = u + NBUF - 2
        if nu < UNITS:
            ns = nu % NBUF
            if wdesc[ns] is not None:
                wdesc[ns].wait()
                wdesc[ns] = None
            gdesc[ns] = start_gather(nu)

        c, b = divmod(u, BATCH)
        if b == 0:
            pdesc[c % 2].wait()
            pdesc[c % 2] = None
        gdesc[slot].wait()

        cur = bufs[slot]
        pb = pbufs[c % 2]

        @plsc.parallel_loop(0, GROUPS, unroll=8)
        def add_group(i):
            r = lax.div(i, G_PER_ROW)
            off = lax.mul(lax.rem(i, G_PER_ROW), 16)
            plsc.addupdate(cur.at[r, pl.ds(off, 16)], pb[r, pl.ds(off, 16)])

        if b == BATCH - 1 and c + 2 < NCHUNK_S:
            pdesc[c % 2] = start_pos(c + 2)

        row0 = b * SEQ_LEN + s_base + c * CHUNK
        wdesc[slot] = pltpu.async_copy(cur, out_hbm.at[pl.ds(row0, CHUNK)],
                                       wsems[slot])

    for d in wdesc:
        if d is not None:
            d.wait()


@jax.jit
def _embed(ids_r, token_table, pos_table):
    mesh = plsc.VectorSubcoreMesh(core_axis_name="c", subcore_axis_name="s")
    k = pl.kernel(
        _body,
        out_type=jax.ShapeDtypeStruct((FLAT, D_MODEL), jnp.float32),
        mesh=mesh,
        scratch_types=(
            [pltpu.VMEM((BATCH, S_PER_W), jnp.int32)]
            + [pltpu.VMEM((CHUNK, D_MODEL), jnp.float32)] * NBUF
            + [pltpu.VMEM((CHUNK, D_MODEL), jnp.float32)] * 2
            + [pltpu.SemaphoreType.DMA] * (2 * NBUF + 2)
        ),
    )
    return k(ids_r, token_table, pos_table)


def kernel(input_ids, token_table, pos_table):
    out = _embed(input_ids.astype(jnp.int32), token_table, pos_table)
    return out.reshape(BATCH, SEQ_LEN, D_MODEL)
